# Initial kernel scaffold; baseline (speedup 1.0000x reference)
import jax
import jax.numpy as jnp
from jax.experimental import pallas as pl


def kernel(feat0, feat1, feat2):
    raise NotImplementedError("write your pallas kernel here")


# speedup vs baseline: 6.6877x; 6.6877x over previous
import numpy as np
import jax
import jax.numpy as jnp
from jax.experimental import pallas as pl

_NC = 80
_REG_MAX = 16
_MAX_DET = 300
_IOU_T = 0.7
_CONF_T = 0.25
_PRE_NMS_K = 2000
_MAX_WH = 7680.0
_LEVELS = (((80, 80), 8.0), ((40, 40), 16.0), ((20, 20), 32.0))
_A = 8400
_APAD = 8448


def _make_aux():
    ax, ay, st = [], [], []
    for (h, w), s in _LEVELS:
        ax.append(np.tile(np.arange(w, dtype=np.float32) + 0.5, h))
        ay.append(np.repeat(np.arange(h, dtype=np.float32) + 0.5, w))
        st.append(np.full(h * w, s, dtype=np.float32))
    aux = np.zeros((8, _APAD), dtype=np.float32)
    aux[0, :_A] = np.concatenate(ax)
    aux[1, :_A] = np.concatenate(ay)
    aux[2, :_A] = np.concatenate(st)
    return aux


_AUX = _make_aux()


def _kern(xbox_ref, scls_ref, aux_ref, out_ref):
    B = xbox_ref.shape[1]
    ax = aux_ref[0:1, :]
    ay = aux_ref[1:2, :]
    st = aux_ref[2:3, :]

    ltrb = []
    for s in range(4):
        rows = [xbox_ref[16 * s + b] for b in range(16)]
        m = rows[0]
        for b in range(1, 16):
            m = jnp.maximum(m, rows[b])
        es = [jnp.exp(r - m) for r in rows]
        se = es[0]
        for b in range(1, 16):
            se = se + es[b]
        acc = es[1] / se
        for b in range(2, 16):
            acc = acc + (es[b] / se) * jnp.float32(b)
        ltrb.append(acc)
    dl, dt, dr, db = ltrb
    x1 = ax - dl
    y1 = ay - dt
    x2 = ax + dr
    y2 = ay + db
    cxA = ((x1 + x2) * 0.5) * st
    cyA = ((y1 + y2) * 0.5) * st
    wA = (x2 - x1) * st
    hA = (y2 - y1) * st

    conf = scls_ref[0]
    clA = jnp.zeros_like(conf)
    for c in range(1, _NC):
        sc = scls_ref[c]
        upd = sc > conf
        conf = jnp.where(upd, sc, conf)
        clA = jnp.where(upd, jnp.float32(c), clA)
    conf = jnp.where(conf > _CONF_T, conf, 0.0)

    bits = jax.lax.bitcast_convert_type(conf, jnp.int32)

    def bs_body(_, carry):
        lo, hi = carry
        mid = lo + ((hi - lo + 1) >> 1)
        cnt = jnp.sum((bits >= mid).astype(jnp.int32), axis=1, keepdims=True)
        ge = cnt >= _PRE_NMS_K
        lo = jnp.where(ge, mid, lo)
        hi = jnp.where(ge, hi, mid - 1)
        return lo, hi

    lo0 = jnp.zeros((B, 1), jnp.int32)
    hi0 = jnp.full((B, 1), 0x3F800000, jnp.int32)
    t, _ = jax.lax.fori_loop(0, 31, bs_body, (lo0, hi0))
    gt = bits > t
    n_gt = jnp.sum(gt.astype(jnp.int32), axis=1, keepdims=True)
    tie = bits == t
    tie_i = tie.astype(jnp.int32)
    lane = jax.lax.broadcasted_iota(jnp.int32, (B, _APAD), 1)
    psum = tie_i
    k = 1
    while k < _APAD:
        psum = psum + jnp.where(lane >= k, jnp.roll(psum, k, axis=1), 0)
        k *= 2
    tie_rank = psum - tie_i
    sel = gt | (tie & (tie_rank < (_PRE_NMS_K - n_gt)))
    live0 = jnp.where(sel, conf, 0.0)

    halfW = wA * 0.5
    halfH = hA * 0.5
    offA = clA * _MAX_WH
    bx0 = (cxA - halfW) + offA
    by0 = (cyA - halfH) + offA
    bx1 = (cxA + halfW) + offA
    by1 = (cyA + halfH) + offA
    areas = (bx1 - bx0) * (by1 - by0)

    def nms_body(i, live):
        m = jnp.max(live, axis=1, keepdims=True)
        ism = live == m
        idx = jnp.min(jnp.where(ism, lane, _APAD), axis=1, keepdims=True)
        onehot = lane == idx

        def pick(arr):
            return jnp.sum(jnp.where(onehot, arr, 0.0), axis=1, keepdims=True)

        cx_s = pick(cxA)
        cy_s = pick(cyA)
        w_s = pick(wA)
        h_s = pick(hA)
        cl_s = pick(clA)
        hw_s = w_s * 0.5
        hh_s = h_s * 0.5
        off_s = cl_s * _MAX_WH
        sx0 = (cx_s - hw_s) + off_s
        sy0 = (cy_s - hh_s) + off_s
        sx1 = (cx_s + hw_s) + off_s
        sy1 = (cy_s + hh_s) + off_s
        ai = (sx1 - sx0) * (sy1 - sy0)
        ix1 = jnp.maximum(sx0, bx0)
        iy1 = jnp.maximum(sy0, by0)
        ix2 = jnp.minimum(sx1, bx1)
        iy2 = jnp.minimum(sy1, by1)
        inter = jnp.maximum(ix2 - ix1, 0.0) * jnp.maximum(iy2 - iy1, 0.0)
        iou = inter / (ai + areas - inter + 1e-7)
        live = jnp.where((iou > _IOU_T) | onehot, 0.0, live)

        valid = m > 0.0
        row = jnp.concatenate([cx_s, cy_s, w_s, h_s, m, cl_s], axis=1)
        row = jnp.where(valid, row, 0.0)
        out_ref[pl.ds(i, 1), :, :] = row[None]
        return live

    jax.lax.fori_loop(0, _MAX_DET, nms_body, live0)


def kernel(feat0, feat1, feat2):
    feats = (feat0, feat1, feat2)
    B = feat0.shape[0]
    no = _NC + 4 * _REG_MAX
    merged = jnp.concatenate([f.reshape(B, no, -1) for f in feats], axis=2)
    xbox = jnp.transpose(merged[:, : 4 * _REG_MAX, :], (1, 0, 2))
    scls = jax.nn.sigmoid(jnp.transpose(merged[:, 4 * _REG_MAX :, :], (1, 0, 2)))
    pad = ((0, 0), (0, 0), (0, _APAD - _A))
    xbox = jnp.pad(xbox, pad)
    scls = jnp.pad(scls, pad)
    out = pl.pallas_call(
        _kern,
        out_shape=jax.ShapeDtypeStruct((_MAX_DET, B, 6), jnp.float32),
    )(xbox, scls, jnp.asarray(_AUX))
    return jnp.transpose(out, (1, 0, 2))


# speedup vs baseline: 8.7320x; 1.3057x over previous
import numpy as np
import jax
import jax.numpy as jnp
from jax.experimental import pallas as pl

_NC = 80
_REG_MAX = 16
_MAX_DET = 300
_IOU_T = 0.7
_CONF_T = 0.25
_PRE_NMS_K = 2000
_MAX_WH = 7680.0
_LEVELS = (((80, 80), 8.0), ((40, 40), 16.0), ((20, 20), 32.0))
_A = 8400
_APAD = 8448
_KC = 2048


def _make_aux():
    ax, ay, st = [], [], []
    for (h, w), s in _LEVELS:
        ax.append(np.tile(np.arange(w, dtype=np.float32) + 0.5, h))
        ay.append(np.repeat(np.arange(h, dtype=np.float32) + 0.5, w))
        st.append(np.full(h * w, s, dtype=np.float32))
    aux = np.zeros((8, _APAD), dtype=np.float32)
    aux[0, :_A] = np.concatenate(ax)
    aux[1, :_A] = np.concatenate(ay)
    aux[2, :_A] = np.concatenate(st)
    return aux


_AUX = _make_aux()


def _kern(xbox_ref, scls_ref, aux_ref, out_ref):
    B = xbox_ref.shape[1]
    ax = aux_ref[0:1, :]
    ay = aux_ref[1:2, :]
    st = aux_ref[2:3, :]

    ltrb = []
    for s in range(4):
        rows = [xbox_ref[16 * s + b] for b in range(16)]
        m = rows[0]
        for b in range(1, 16):
            m = jnp.maximum(m, rows[b])
        es = [jnp.exp(r - m) for r in rows]
        se = es[0]
        for b in range(1, 16):
            se = se + es[b]
        acc = es[1] / se
        for b in range(2, 16):
            acc = acc + (es[b] / se) * jnp.float32(b)
        ltrb.append(acc)
    dl, dt, dr, db = ltrb
    x1 = ax - dl
    y1 = ay - dt
    x2 = ax + dr
    y2 = ay + db
    cxA = ((x1 + x2) * 0.5) * st
    cyA = ((y1 + y2) * 0.5) * st
    wA = (x2 - x1) * st
    hA = (y2 - y1) * st

    conf = scls_ref[0]
    clA = jnp.zeros_like(conf)
    for c in range(1, _NC):
        sc = scls_ref[c]
        upd = sc > conf
        conf = jnp.where(upd, sc, conf)
        clA = jnp.where(upd, jnp.float32(c), clA)
    conf = jnp.where(conf > _CONF_T, conf, 0.0)

    bits = jax.lax.bitcast_convert_type(conf, jnp.int32)

    def bs_body(_, carry):
        lo, hi = carry
        mid = lo + ((hi - lo + 1) >> 1)
        cnt = jnp.sum((bits >= mid).astype(jnp.int32), axis=1, keepdims=True)
        ge = cnt >= _PRE_NMS_K
        lo = jnp.where(ge, mid, lo)
        hi = jnp.where(ge, hi, mid - 1)
        return lo, hi

    lo0 = jnp.zeros((B, 1), jnp.int32)
    hi0 = jnp.full((B, 1), 0x3F800000, jnp.int32)
    t, _ = jax.lax.fori_loop(0, 31, bs_body, (lo0, hi0))
    gt = bits > t
    n_gt = jnp.sum(gt.astype(jnp.int32), axis=1, keepdims=True)
    tie = bits == t
    tie_i = tie.astype(jnp.int32)
    lane = jax.lax.broadcasted_iota(jnp.int32, (B, _APAD), 1)

    def excl_cumsum(x):
        ps = x
        k = 1
        while k < _APAD:
            ps = ps + jnp.where(lane >= k, jnp.roll(ps, k, axis=1), 0)
            k *= 2
        return ps - x

    tie_rank = excl_cumsum(tie_i)
    sel = gt | (tie & (tie_rank < (_PRE_NMS_K - n_gt)))
    live0 = jnp.where(sel, conf, 0.0)

    sel_i = sel.astype(jnp.int32)
    s = lane - excl_cumsum(sel_i)
    payload = [live0, cxA, cyA, wA, hA, clA]
    for k in range(14):
        step = 1 << k
        moving_i = sel_i * ((s >> k) & 1)
        incoming_i = jnp.roll(moving_i, -step, axis=1)
        incoming = incoming_i == 1
        payload = [
            jnp.where(incoming, jnp.roll(a, -step, axis=1), a) for a in payload
        ]
        s = jnp.where(incoming, jnp.roll(s, -step, axis=1), s)
        sel_i = sel_i - moving_i + incoming_i
    live_c, cxA, cyA, wA, hA, clA = [a[:, :_KC] for a in payload]
    live0 = jnp.where(sel_i[:, :_KC] == 1, live_c, 0.0)
    lane_c = jax.lax.broadcasted_iota(jnp.int32, (B, _KC), 1)

    halfW = wA * 0.5
    halfH = hA * 0.5
    offA = clA * _MAX_WH
    bx0 = (cxA - halfW) + offA
    by0 = (cyA - halfH) + offA
    bx1 = (cxA + halfW) + offA
    by1 = (cyA + halfH) + offA
    areas = (bx1 - bx0) * (by1 - by0)

    def nms_body(i, live):
        m = jnp.max(live, axis=1, keepdims=True)
        ism = live == m
        idx = jnp.min(jnp.where(ism, lane_c, _KC), axis=1, keepdims=True)
        onehot = lane_c == idx

        def pick(arr):
            return jnp.sum(jnp.where(onehot, arr, 0.0), axis=1, keepdims=True)

        cx_s = pick(cxA)
        cy_s = pick(cyA)
        w_s = pick(wA)
        h_s = pick(hA)
        cl_s = pick(clA)
        hw_s = w_s * 0.5
        hh_s = h_s * 0.5
        off_s = cl_s * _MAX_WH
        sx0 = (cx_s - hw_s) + off_s
        sy0 = (cy_s - hh_s) + off_s
        sx1 = (cx_s + hw_s) + off_s
        sy1 = (cy_s + hh_s) + off_s
        ai = (sx1 - sx0) * (sy1 - sy0)
        ix1 = jnp.maximum(sx0, bx0)
        iy1 = jnp.maximum(sy0, by0)
        ix2 = jnp.minimum(sx1, bx1)
        iy2 = jnp.minimum(sy1, by1)
        inter = jnp.maximum(ix2 - ix1, 0.0) * jnp.maximum(iy2 - iy1, 0.0)
        iou = inter / (ai + areas - inter + 1e-7)
        live = jnp.where((iou > _IOU_T) | onehot, 0.0, live)

        valid = m > 0.0
        row = jnp.concatenate([cx_s, cy_s, w_s, h_s, m, cl_s], axis=1)
        row = jnp.where(valid, row, 0.0)
        out_ref[pl.ds(i, 1), :, :] = row[None]
        return live

    jax.lax.fori_loop(0, _MAX_DET, nms_body, live0)


def kernel(feat0, feat1, feat2):
    feats = (feat0, feat1, feat2)
    B = feat0.shape[0]
    no = _NC + 4 * _REG_MAX
    merged = jnp.concatenate([f.reshape(B, no, -1) for f in feats], axis=2)
    xbox = jnp.transpose(merged[:, : 4 * _REG_MAX, :], (1, 0, 2))
    scls = jax.nn.sigmoid(jnp.transpose(merged[:, 4 * _REG_MAX :, :], (1, 0, 2)))
    pad = ((0, 0), (0, 0), (0, _APAD - _A))
    xbox = jnp.pad(xbox, pad)
    scls = jnp.pad(scls, pad)
    out = pl.pallas_call(
        _kern,
        out_shape=jax.ShapeDtypeStruct((_MAX_DET, B, 6), jnp.float32),
    )(xbox, scls, jnp.asarray(_AUX))
    return jnp.transpose(out, (1, 0, 2))


# speedup vs baseline: 10.9168x; 1.2502x over previous
import numpy as np
import jax
import jax.numpy as jnp
from jax.experimental import pallas as pl

_NC = 80
_REG_MAX = 16
_MAX_DET = 300
_IOU_T = 0.7
_CONF_T = 0.25
_PRE_NMS_K = 2000
_MAX_WH = 7680.0
_LEVELS = (((80, 80), 8.0), ((40, 40), 16.0), ((20, 20), 32.0))
_A = 8400
_LV_HW = (6400, 1600, 400)
_LV_PAD = (6400, 1664, 512)
_APAD = sum(_LV_PAD)
_KC = 2048


def _make_aux():
    aux = np.zeros((8, _APAD), dtype=np.float32)
    o = 0
    for ((h, w), s), hwp in zip(_LEVELS, _LV_PAD):
        aux[0, o : o + h * w] = np.tile(np.arange(w, dtype=np.float32) + 0.5, h)
        aux[1, o : o + h * w] = np.repeat(np.arange(h, dtype=np.float32) + 0.5, w)
        aux[2, o : o + h * w] = s
        o += hwp
    return aux


_AUX = _make_aux()


def _kern(xb0_ref, xb1_ref, xb2_ref, sc0_ref, sc1_ref, sc2_ref, aux_ref, out_ref):
    B = xb0_ref.shape[0]
    ax = aux_ref[0:1, :]
    ay = aux_ref[1:2, :]
    st = aux_ref[2:3, :]

    ltrb_lv = []
    conf_lv = []
    cl_lv = []
    for xb_ref, sc_ref in ((xb0_ref, sc0_ref), (xb1_ref, sc1_ref), (xb2_ref, sc2_ref)):
        ltrb = []
        for s in range(4):
            rows = [xb_ref[:, 16 * s + b, :] for b in range(16)]
            m = rows[0]
            for b in range(1, 16):
                m = jnp.maximum(m, rows[b])
            es = [jnp.exp(r - m) for r in rows]
            se = es[0]
            for b in range(1, 16):
                se = se + es[b]
            acc = es[1] / se
            for b in range(2, 16):
                acc = acc + (es[b] / se) * jnp.float32(b)
            ltrb.append(acc)
        ltrb_lv.append(ltrb)

        conf = sc_ref[:, 0, :]
        cl = jnp.zeros_like(conf)
        for c in range(1, _NC):
            sc = sc_ref[:, c, :]
            upd = sc > conf
            conf = jnp.where(upd, sc, conf)
            cl = jnp.where(upd, jnp.float32(c), cl)
        conf_lv.append(conf)
        cl_lv.append(cl)

    def cat(vals):
        return jnp.concatenate(list(vals), axis=1)

    dl, dt, dr, db = (cat(lv[s] for lv in ltrb_lv) for s in range(4))
    conf = cat(conf_lv)
    clA = cat(cl_lv)
    x1 = ax - dl
    y1 = ay - dt
    x2 = ax + dr
    y2 = ay + db
    cxA = ((x1 + x2) * 0.5) * st
    cyA = ((y1 + y2) * 0.5) * st
    wA = (x2 - x1) * st
    hA = (y2 - y1) * st

    conf = jnp.where(conf > _CONF_T, conf, 0.0)

    bits = jax.lax.bitcast_convert_type(conf, jnp.int32)

    def bs_body(_, carry):
        lo, hi = carry
        mid = lo + ((hi - lo + 1) >> 1)
        cnt = jnp.sum((bits >= mid).astype(jnp.int32), axis=1, keepdims=True)
        ge = cnt >= _PRE_NMS_K
        lo = jnp.where(ge, mid, lo)
        hi = jnp.where(ge, hi, mid - 1)
        return lo, hi

    lo0 = jnp.zeros((B, 1), jnp.int32)
    hi0 = jnp.full((B, 1), 0x3F800000, jnp.int32)
    t, _ = jax.lax.fori_loop(0, 31, bs_body, (lo0, hi0))
    gt = bits > t
    n_gt = jnp.sum(gt.astype(jnp.int32), axis=1, keepdims=True)
    tie = bits == t
    tie_i = tie.astype(jnp.int32)
    lane = jax.lax.broadcasted_iota(jnp.int32, (B, _APAD), 1)

    def excl_cumsum(x):
        ps = x
        k = 1
        while k < _APAD:
            ps = ps + jnp.where(lane >= k, jnp.roll(ps, k, axis=1), 0)
            k *= 2
        return ps - x

    tie_rank = excl_cumsum(tie_i)
    sel = gt | (tie & (tie_rank < (_PRE_NMS_K - n_gt)))
    live0 = jnp.where(sel, conf, 0.0)

    sel_i = sel.astype(jnp.int32)
    s = lane - excl_cumsum(sel_i)
    payload = [live0, cxA, cyA, wA, hA, clA]
    for k in range(14):
        step = 1 << k
        moving_i = sel_i * ((s >> k) & 1)
        incoming_i = jnp.roll(moving_i, -step, axis=1)
        incoming = incoming_i == 1
        payload = [
            jnp.where(incoming, jnp.roll(a, -step, axis=1), a) for a in payload
        ]
        s = jnp.where(incoming, jnp.roll(s, -step, axis=1), s)
        sel_i = sel_i - moving_i + incoming_i
    live_c, cxA, cyA, wA, hA, clA = [a[:, :_KC] for a in payload]
    live0 = jnp.where(sel_i[:, :_KC] == 1, live_c, 0.0)
    lane_c = jax.lax.broadcasted_iota(jnp.int32, (B, _KC), 1)

    halfW = wA * 0.5
    halfH = hA * 0.5
    offA = clA * _MAX_WH
    bx0 = (cxA - halfW) + offA
    by0 = (cyA - halfH) + offA
    bx1 = (cxA + halfW) + offA
    by1 = (cyA + halfH) + offA
    areas = (bx1 - bx0) * (by1 - by0)

    def nms_body(i, live):
        m = jnp.max(live, axis=1, keepdims=True)
        ism = live == m
        idx = jnp.min(jnp.where(ism, lane_c, _KC), axis=1, keepdims=True)
        onehot = lane_c == idx

        def pick(arr):
            return jnp.sum(jnp.where(onehot, arr, 0.0), axis=1, keepdims=True)

        cx_s = pick(cxA)
        cy_s = pick(cyA)
        w_s = pick(wA)
        h_s = pick(hA)
        cl_s = pick(clA)
        hw_s = w_s * 0.5
        hh_s = h_s * 0.5
        off_s = cl_s * _MAX_WH
        sx0 = (cx_s - hw_s) + off_s
        sy0 = (cy_s - hh_s) + off_s
        sx1 = (cx_s + hw_s) + off_s
        sy1 = (cy_s + hh_s) + off_s
        ai = (sx1 - sx0) * (sy1 - sy0)
        ix1 = jnp.maximum(sx0, bx0)
        iy1 = jnp.maximum(sy0, by0)
        ix2 = jnp.minimum(sx1, bx1)
        iy2 = jnp.minimum(sy1, by1)
        inter = jnp.maximum(ix2 - ix1, 0.0) * jnp.maximum(iy2 - iy1, 0.0)
        iou = inter / (ai + areas - inter + 1e-7)
        live = jnp.where((iou > _IOU_T) | onehot, 0.0, live)

        valid = m > 0.0
        row = jnp.concatenate([cx_s, cy_s, w_s, h_s, m, cl_s], axis=1)
        row = jnp.where(valid, row, 0.0)
        out_ref[pl.ds(i, 1), :, :] = row[None]
        return live

    jax.lax.fori_loop(0, _MAX_DET, nms_body, live0)


def kernel(feat0, feat1, feat2):
    feats = (feat0, feat1, feat2)
    B = feat0.shape[0]
    no = _NC + 4 * _REG_MAX
    rs = [f.reshape(B, no, -1) for f in feats]
    pads = [((0, 0), (0, 0), (0, p - hw)) for hw, p in zip(_LV_HW, _LV_PAD)]
    xbs = [jnp.pad(r[:, : 4 * _REG_MAX, :], p) for r, p in zip(rs, pads)]
    scs = [
        jnp.pad(jax.nn.sigmoid(r[:, 4 * _REG_MAX :, :]), p)
        for r, p in zip(rs, pads)
    ]
    out = pl.pallas_call(
        _kern,
        out_shape=jax.ShapeDtypeStruct((_MAX_DET, B, 6), jnp.float32),
    )(*xbs, *scs, jnp.asarray(_AUX))
    return jnp.transpose(out, (1, 0, 2))


# speedup vs baseline: 11.5768x; 1.0605x over previous
import numpy as np
import jax
import jax.numpy as jnp
from jax.experimental import pallas as pl

_NC = 80
_REG_MAX = 16
_MAX_DET = 300
_IOU_T = 0.7
_CONF_T = 0.25
_PRE_NMS_K = 2000
_MAX_WH = 7680.0
_LEVELS = (((80, 80), 8.0), ((40, 40), 16.0), ((20, 20), 32.0))
_A = 8400
_LV_HW = (6400, 1600, 400)
_LV_PAD = (6400, 1664, 512)
_APAD = sum(_LV_PAD)
_KC = 2048


def _make_aux():
    aux = np.zeros((8, _APAD), dtype=np.float32)
    o = 0
    for ((h, w), s), hwp in zip(_LEVELS, _LV_PAD):
        aux[0, o : o + h * w] = np.tile(np.arange(w, dtype=np.float32) + 0.5, h)
        aux[1, o : o + h * w] = np.repeat(np.arange(h, dtype=np.float32) + 0.5, w)
        aux[2, o : o + h * w] = s
        o += hwp
    return aux


_AUX = _make_aux()


def _kern(xb0_ref, xb1_ref, xb2_ref, sc0_ref, sc1_ref, sc2_ref, aux_ref, out_ref):
    B = xb0_ref.shape[0]
    ax = aux_ref[0:1, :]
    ay = aux_ref[1:2, :]
    st = aux_ref[2:3, :]

    ltrb_lv = []
    conf_lv = []
    cl_lv = []
    for xb_ref, sc_ref in ((xb0_ref, sc0_ref), (xb1_ref, sc1_ref), (xb2_ref, sc2_ref)):
        ltrb = []
        for s in range(4):
            rows = [xb_ref[:, 16 * s + b, :] for b in range(16)]
            m = rows[0]
            for b in range(1, 16):
                m = jnp.maximum(m, rows[b])
            es = [jnp.exp(r - m) for r in rows]
            se = es[0]
            for b in range(1, 16):
                se = se + es[b]
            acc = es[1] / se
            for b in range(2, 16):
                acc = acc + (es[b] / se) * jnp.float32(b)
            ltrb.append(acc)
        ltrb_lv.append(ltrb)

        block = sc_ref[:, :, :]
        conf_lv.append(jnp.max(block, axis=1))
        cl_lv.append(jnp.argmax(block, axis=1).astype(jnp.float32))

    def cat(vals):
        return jnp.concatenate(list(vals), axis=1)

    dl, dt, dr, db = (cat(lv[s] for lv in ltrb_lv) for s in range(4))
    conf = cat(conf_lv)
    clA = cat(cl_lv)
    x1 = ax - dl
    y1 = ay - dt
    x2 = ax + dr
    y2 = ay + db
    cxA = ((x1 + x2) * 0.5) * st
    cyA = ((y1 + y2) * 0.5) * st
    wA = (x2 - x1) * st
    hA = (y2 - y1) * st

    conf = jnp.where(conf > _CONF_T, conf, 0.0)

    bits = jax.lax.bitcast_convert_type(conf, jnp.int32)

    def bs_body(_, carry):
        lo, hi = carry
        mid = lo + ((hi - lo + 1) >> 1)
        cnt = jnp.sum((bits >= mid).astype(jnp.int32), axis=1, keepdims=True)
        ge = cnt >= _PRE_NMS_K
        lo = jnp.where(ge, mid, lo)
        hi = jnp.where(ge, hi, mid - 1)
        return lo, hi

    lo0 = jnp.zeros((B, 1), jnp.int32)
    hi0 = jnp.full((B, 1), 0x3F800000, jnp.int32)
    t, _ = jax.lax.fori_loop(0, 31, bs_body, (lo0, hi0))
    gt = bits > t
    n_gt = jnp.sum(gt.astype(jnp.int32), axis=1, keepdims=True)
    tie = bits == t
    tie_i = tie.astype(jnp.int32)
    lane = jax.lax.broadcasted_iota(jnp.int32, (B, _APAD), 1)

    def excl_cumsum(x):
        ps = x
        k = 1
        while k < _APAD:
            ps = ps + jnp.where(lane >= k, jnp.roll(ps, k, axis=1), 0)
            k *= 2
        return ps - x

    tie_rank = excl_cumsum(tie_i)
    sel = gt | (tie & (tie_rank < (_PRE_NMS_K - n_gt)))
    live0 = jnp.where(sel, conf, 0.0)

    sel_i = sel.astype(jnp.int32)
    s = lane - excl_cumsum(sel_i)
    payload = [live0, cxA, cyA, wA, hA, clA]
    for k in range(14):
        step = 1 << k
        moving_i = sel_i * ((s >> k) & 1)
        incoming_i = jnp.roll(moving_i, -step, axis=1)
        incoming = incoming_i == 1
        payload = [
            jnp.where(incoming, jnp.roll(a, -step, axis=1), a) for a in payload
        ]
        s = jnp.where(incoming, jnp.roll(s, -step, axis=1), s)
        sel_i = sel_i - moving_i + incoming_i
    live_c, cxA, cyA, wA, hA, clA = [a[:, :_KC] for a in payload]
    live0 = jnp.where(sel_i[:, :_KC] == 1, live_c, 0.0)
    lane_c = jax.lax.broadcasted_iota(jnp.int32, (B, _KC), 1)

    halfW = wA * 0.5
    halfH = hA * 0.5
    offA = clA * _MAX_WH
    bx0 = (cxA - halfW) + offA
    by0 = (cyA - halfH) + offA
    bx1 = (cxA + halfW) + offA
    by1 = (cyA + halfH) + offA
    areas = (bx1 - bx0) * (by1 - by0)

    lane16 = lane_c << 16
    _BIGK = jnp.int32(0x7FFFFFFF)

    def keys_of(arr):
        b = jax.lax.bitcast_convert_type(arr, jnp.int32)
        hi = jax.lax.shift_right_logical(b, 16)
        lo = b & 0xFFFF
        return lane16 | hi, lane16 | lo

    kcx_h, kcx_l = keys_of(cxA)
    kcy_h, kcy_l = keys_of(cyA)
    kw_h, kw_l = keys_of(wA)
    kh_h, kh_l = keys_of(hA)
    kcl = (lane_c << 8) | clA.astype(jnp.int32)

    def nms_body(i, live):
        m = jnp.max(live, axis=1, keepdims=True)
        ism = live == m

        def kmin(key):
            return jnp.min(jnp.where(ism, key, _BIGK), axis=1, keepdims=True)

        def unpack(kh, kl):
            b = ((kh & 0xFFFF) << 16) | (kl & 0xFFFF)
            return jax.lax.bitcast_convert_type(b, jnp.float32)

        k0 = kmin(kcl)
        cx_s = unpack(kmin(kcx_h), kmin(kcx_l))
        cy_s = unpack(kmin(kcy_h), kmin(kcy_l))
        w_s = unpack(kmin(kw_h), kmin(kw_l))
        h_s = unpack(kmin(kh_h), kmin(kh_l))
        cl_s = (k0 & 255).astype(jnp.float32)
        idx = jax.lax.shift_right_logical(k0, 8)
        onehot = lane_c == idx
        hw_s = w_s * 0.5
        hh_s = h_s * 0.5
        off_s = cl_s * _MAX_WH
        sx0 = (cx_s - hw_s) + off_s
        sy0 = (cy_s - hh_s) + off_s
        sx1 = (cx_s + hw_s) + off_s
        sy1 = (cy_s + hh_s) + off_s
        ai = (sx1 - sx0) * (sy1 - sy0)
        ix1 = jnp.maximum(sx0, bx0)
        iy1 = jnp.maximum(sy0, by0)
        ix2 = jnp.minimum(sx1, bx1)
        iy2 = jnp.minimum(sy1, by1)
        inter = jnp.maximum(ix2 - ix1, 0.0) * jnp.maximum(iy2 - iy1, 0.0)
        iou = inter / (ai + areas - inter + 1e-7)
        live = jnp.where((iou > _IOU_T) | onehot, 0.0, live)

        valid = m > 0.0
        row = jnp.concatenate([cx_s, cy_s, w_s, h_s, m, cl_s], axis=1)
        row = jnp.where(valid, row, 0.0)
        out_ref[pl.ds(i, 1), :, :] = row[None]
        return live

    jax.lax.fori_loop(0, _MAX_DET, nms_body, live0)


def kernel(feat0, feat1, feat2):
    feats = (feat0, feat1, feat2)
    B = feat0.shape[0]
    no = _NC + 4 * _REG_MAX
    rs = [f.reshape(B, no, -1) for f in feats]
    pads = [((0, 0), (0, 0), (0, p - hw)) for hw, p in zip(_LV_HW, _LV_PAD)]
    xbs = [jnp.pad(r[:, : 4 * _REG_MAX, :], p) for r, p in zip(rs, pads)]
    scs = [
        jnp.pad(jax.nn.sigmoid(r[:, 4 * _REG_MAX :, :]), p)
        for r, p in zip(rs, pads)
    ]
    out = pl.pallas_call(
        _kern,
        out_shape=jax.ShapeDtypeStruct((_MAX_DET, B, 6), jnp.float32),
    )(*xbs, *scs, jnp.asarray(_AUX))
    return jnp.transpose(out, (1, 0, 2))
